# Initial kernel scaffold; baseline (speedup 1.0000x reference)
import functools

import jax
import jax.numpy as jnp
from jax import lax
from jax.experimental import pallas as pl
from jax.experimental.pallas import tpu as pltpu
from jax.experimental.pallas import tpu_sc as plsc

N = 10000
E = 320000
F = 128
C_OUT = 40

NC = 2
NS = 16
NW = NC * NS
EPW = E // NW
CH = 80
NCHUNK = EPW // CH
RPT = N // NS
ZCH = 125
NP = 10240
DPT = NP // NS

_mesh = plsc.VectorSubcoreMesh(core_axis_name="c", subcore_axis_name="s")


def _fill_zero_rows(buf, nrows, ncol16):
    zf = jnp.zeros((16,), jnp.float32)

    def body(r, carry):
        for j in range(ncol16):
            buf[r, pl.ds(j * 16, 16)] = zf
        return carry

    lax.fori_loop(0, nrows, body, 0)


@functools.partial(
    pl.kernel,
    out_type=jax.ShapeDtypeStruct((NC, 2, NP, 16), jnp.float32),
    mesh=_mesh,
    scratch_types=[
        pltpu.VMEM((NCHUNK, CH), jnp.int32),
        pltpu.VMEM((NCHUNK, CH), jnp.int32),
        pltpu.VMEM((CH, 16), jnp.float32),
        pltpu.VMEM((DPT, 16), jnp.float32),
        pltpu.VMEM_SHARED((NP, 16), jnp.float32),
        pltpu.VMEM_SHARED((NP, 16), jnp.float32),
    ],
)
def _deg(src_hbm, dst_hbm, out_hbm, idx_s, idx_d, ones_v, stage_v,
         dego_sh, degi_sh):
    cid = lax.axis_index("c")
    sid = lax.axis_index("s")
    wid = cid * NS + sid

    pltpu.sync_copy(src_hbm.at[wid], idx_s)
    pltpu.sync_copy(dst_hbm.at[wid], idx_d)

    onehot = jnp.where(lax.iota(jnp.int32, 16) == 0,
                       jnp.full((16,), 1.0, jnp.float32),
                       jnp.zeros((16,), jnp.float32))

    def fill(r, carry):
        ones_v[r, pl.ds(0, 16)] = onehot
        return carry

    lax.fori_loop(0, CH, fill, 0)
    _fill_zero_rows(stage_v, DPT, 1)

    base = sid * DPT
    pltpu.sync_copy(stage_v, dego_sh.at[pl.ds(base, DPT), :])
    pltpu.sync_copy(stage_v, degi_sh.at[pl.ds(base, DPT), :])
    plsc.subcore_barrier()

    def step(j, carry):
        pltpu.sync_copy(ones_v, dego_sh.at[idx_s.at[j]], add=True)
        pltpu.sync_copy(ones_v, degi_sh.at[idx_d.at[j]], add=True)
        return carry

    lax.fori_loop(0, NCHUNK, step, 0)
    plsc.subcore_barrier()

    pltpu.sync_copy(dego_sh.at[pl.ds(base, DPT), :], stage_v)
    pltpu.sync_copy(stage_v, out_hbm.at[cid, 0, pl.ds(base, DPT), :])
    pltpu.sync_copy(degi_sh.at[pl.ds(base, DPT), :], stage_v)
    pltpu.sync_copy(stage_v, out_hbm.at[cid, 1, pl.ds(base, DPT), :])


def _make_agg(d):
    @functools.partial(
        pl.kernel,
        out_type=jax.ShapeDtypeStruct((NC, N, d), jnp.float32),
        mesh=_mesh,
        scratch_types=[
            pltpu.VMEM((NCHUNK, CH), jnp.int32),
            pltpu.VMEM((NCHUNK, CH), jnp.int32),
            pltpu.VMEM((CH, d), jnp.float32),
            pltpu.VMEM((ZCH, d), jnp.float32),
            pltpu.VMEM_SHARED((N, d), jnp.float32),
            pltpu.SemaphoreType.DMA,
        ],
    )
    def agg(x_hbm, src_hbm, dst_hbm, out_hbm, idx_s, idx_d, rows_v, stage_v,
            acc_sh, sem):
        cid = lax.axis_index("c")
        sid = lax.axis_index("s")
        wid = cid * NS + sid

        pltpu.sync_copy(src_hbm.at[wid], idx_s)
        pltpu.sync_copy(dst_hbm.at[wid], idx_d)

        _fill_zero_rows(stage_v, ZCH, d // 16)
        base = sid * RPT
        for k in range(RPT // ZCH):
            pltpu.sync_copy(stage_v, acc_sh.at[pl.ds(base + k * ZCH, ZCH), :])
        plsc.subcore_barrier()

        def step(j, carry):
            pltpu.async_copy(x_hbm.at[idx_s.at[j]], rows_v, sem).wait()
            pltpu.sync_copy(rows_v, acc_sh.at[idx_d.at[j]], add=True)
            return carry

        lax.fori_loop(0, NCHUNK, step, 0)
        plsc.subcore_barrier()

        for k in range(RPT // ZCH):
            off = base + k * ZCH
            pltpu.sync_copy(acc_sh.at[pl.ds(off, ZCH), :], stage_v)
            pltpu.sync_copy(stage_v, out_hbm.at[cid, pl.ds(off, ZCH), :])

    return agg


_agg128 = _make_agg(F)
_agg64 = _make_agg(64)


_R = 1000


def _norm_body(d_ref, ns_ref, nd_ref):
    do = d_ref[0:1, :] + d_ref[2:3, :]
    di = d_ref[1:2, :] + d_ref[3:4, :]
    ns_ref[...] = lax.rsqrt(jnp.maximum(do, 1.0))
    nd_ref[...] = lax.rsqrt(jnp.maximum(di, 1.0))


_norm = pl.pallas_call(
    _norm_body,
    out_shape=(jax.ShapeDtypeStruct((1, NP), jnp.float32),
               jax.ShapeDtypeStruct((1, NP), jnp.float32)),
)


def _mm1_body(ns_ref, h_ref, w_ref, o_ref):
    o_ref[...] = jnp.dot(ns_ref[...] * h_ref[...], w_ref[...],
                         preferred_element_type=jnp.float32)


_mm1 = pl.pallas_call(
    _mm1_body,
    grid=(N // _R,),
    in_specs=[
        pl.BlockSpec((_R, 1), lambda i: (i, 0)),
        pl.BlockSpec((_R, F), lambda i: (i, 0)),
        pl.BlockSpec((F, F), lambda i: (0, 0)),
    ],
    out_specs=pl.BlockSpec((_R, F), lambda i: (i, 0)),
    out_shape=jax.ShapeDtypeStruct((N, F), jnp.float32),
)


def _make_layer(dout):
    def body(a0_ref, a1_ref, nd_ref, ns_ref, w_ref, o_ref):
        hp = jnp.maximum(nd_ref[...] * (a0_ref[...] + a1_ref[...]), 0.0)
        o_ref[...] = jnp.dot(ns_ref[...] * hp, w_ref[...],
                             preferred_element_type=jnp.float32)

    return pl.pallas_call(
        body,
        grid=(N // _R,),
        in_specs=[
            pl.BlockSpec((_R, F), lambda i: (i, 0)),
            pl.BlockSpec((_R, F), lambda i: (i, 0)),
            pl.BlockSpec((_R, 1), lambda i: (i, 0)),
            pl.BlockSpec((_R, 1), lambda i: (i, 0)),
            pl.BlockSpec((F, dout), lambda i: (0, 0)),
        ],
        out_specs=pl.BlockSpec((_R, dout), lambda i: (i, 0)),
        out_shape=jax.ShapeDtypeStruct((N, dout), jnp.float32),
    )


_layer128 = _make_layer(F)
_layer64 = _make_layer(64)


def _final_body(a0_ref, a1_ref, nd_ref, o_ref):
    o_ref[...] = jnp.maximum(nd_ref[...] * (a0_ref[...] + a1_ref[...]), 0.0)


_final = pl.pallas_call(
    _final_body,
    grid=(N // _R,),
    in_specs=[
        pl.BlockSpec((_R, 64), lambda i: (i, 0)),
        pl.BlockSpec((_R, 64), lambda i: (i, 0)),
        pl.BlockSpec((_R, 1), lambda i: (i, 0)),
    ],
    out_specs=pl.BlockSpec((_R, 64), lambda i: (i, 0)),
    out_shape=jax.ShapeDtypeStruct((N, 64), jnp.float32),
)


def kernel(h, edge_index, W1, W2, W3):
    src = edge_index[0].reshape(NW, NCHUNK, CH)
    dst = edge_index[1].reshape(NW, NCHUNK, CH)

    degp = _deg(src, dst)
    d4 = degp[..., 0].reshape(4, NP)
    ns_row, nd_row = _norm(d4)
    ns = ns_row.reshape(NP, 1)[:N]
    nd = nd_row.reshape(NP, 1)[:N]

    x1 = _mm1(ns, h, W1)
    a1 = _agg128(x1, src, dst)
    x2 = _layer128(a1[0], a1[1], nd, ns, W2)
    a2 = _agg128(x2, src, dst)
    W3p = jnp.pad(W3, ((0, 0), (0, 64 - C_OUT)))
    x3 = _layer64(a2[0], a2[1], nd, ns, W3p)
    a3 = _agg64(x3, src, dst)
    out = _final(a3[0], a3[1], nd)
    return out[:, :C_OUT]


# speedup vs baseline: 6.8607x; 6.8607x over previous
import functools

import jax
import jax.numpy as jnp
from jax import lax
from jax.experimental import pallas as pl
from jax.experimental.pallas import tpu as pltpu
from jax.experimental.pallas import tpu_sc as plsc

N = 10000
E = 320000
F = 128
C_OUT = 40

NC = 2
NS = 16
NW = NC * NS
EPW = E // NW
CH = 80
NCHUNK = EPW // CH
SUB = 25
NSUPER = NCHUNK // SUB
NP = 10240
NA = NP
RPT = NA // NS
ZCH = 64
DPT = NP // NS

@functools.cache
def _mesh():
    return plsc.VectorSubcoreMesh(core_axis_name="c", subcore_axis_name="s",
                                  num_cores=NC, num_subcores=NS)


def _fill_zero_rows(buf, nrows, ncol16):
    zf = jnp.zeros((16,), jnp.float32)

    def body(r, carry):
        for j in range(ncol16):
            buf[r, pl.ds(j * 16, 16)] = zf
        return carry

    lax.fori_loop(0, nrows, body, 0)


DW = 8


@functools.cache
def _make_deg():
    return functools.partial(
        pl.kernel,
        out_type=jax.ShapeDtypeStruct((NC, 2, NP, DW), jnp.float32),
        mesh=_mesh(),
        compiler_params=pltpu.CompilerParams(use_tc_tiling_on_sc=False),
        scratch_types=[
            pltpu.VMEM((SUB, CH), jnp.int32),
            pltpu.VMEM((SUB, CH), jnp.int32),
            pltpu.VMEM((CH, DW), jnp.float32),
            pltpu.VMEM((DPT, DW), jnp.float32),
            pltpu.VMEM_SHARED((NP, DW), jnp.float32),
            pltpu.VMEM_SHARED((NP, DW), jnp.float32),
        ],
    )(_deg_body)


def _deg_body(src_hbm, dst_hbm, ones_hbm, zero_hbm, out_hbm,
              idx_s, idx_d, ones_v, stage_v, dego_sh, degi_sh):
    cid = lax.axis_index("c")
    sid = lax.axis_index("s")
    wid = cid * NS + sid

    pltpu.sync_copy(ones_hbm, ones_v)
    pltpu.sync_copy(zero_hbm, stage_v)

    base = sid * DPT
    pltpu.sync_copy(stage_v, dego_sh.at[pl.ds(base, DPT), :])
    pltpu.sync_copy(stage_v, degi_sh.at[pl.ds(base, DPT), :])
    plsc.subcore_barrier()

    def super_step(g, carry):
        pltpu.sync_copy(src_hbm.at[wid, g], idx_s)
        pltpu.sync_copy(dst_hbm.at[wid, g], idx_d)

        def step(j, c2):
            pltpu.sync_copy(ones_v, dego_sh.at[idx_s.at[j]], add=True)
            pltpu.sync_copy(ones_v, degi_sh.at[idx_d.at[j]], add=True)
            return c2

        lax.fori_loop(0, SUB, step, 0)
        return carry

    lax.fori_loop(0, NSUPER, super_step, 0)
    plsc.subcore_barrier()

    pltpu.sync_copy(dego_sh.at[pl.ds(base, DPT), :], stage_v)
    pltpu.sync_copy(stage_v, out_hbm.at[cid, 0, pl.ds(base, DPT), :])
    pltpu.sync_copy(degi_sh.at[pl.ds(base, DPT), :], stage_v)
    pltpu.sync_copy(stage_v, out_hbm.at[cid, 1, pl.ds(base, DPT), :])


@functools.cache
def _make_agg(d):
    @functools.partial(
        pl.kernel,
        out_type=jax.ShapeDtypeStruct((NC, NA, d), jnp.float32),
        mesh=_mesh(),
        scratch_types=[
            pltpu.VMEM((SUB, CH), jnp.int32),
            pltpu.VMEM((SUB, CH), jnp.int32),
            pltpu.VMEM((CH, d), jnp.float32),
            pltpu.VMEM((ZCH, d), jnp.float32),
            pltpu.VMEM_SHARED((NA, d), jnp.float32),
            pltpu.SemaphoreType.DMA,
        ],
    )
    def agg(x_hbm, src_hbm, dst_hbm, out_hbm, idx_s, idx_d, rows_v, stage_v,
            acc_sh, sem):
        cid = lax.axis_index("c")
        sid = lax.axis_index("s")
        wid = cid * NS + sid

        _fill_zero_rows(stage_v, ZCH, d // 16)
        base = sid * RPT
        for k in range(RPT // ZCH):
            pltpu.sync_copy(stage_v, acc_sh.at[pl.ds(base + k * ZCH, ZCH), :])
        plsc.subcore_barrier()

        def super_step(g, carry):
            pltpu.sync_copy(src_hbm.at[wid, g], idx_s)
            pltpu.sync_copy(dst_hbm.at[wid, g], idx_d)

            def step(j, c2):
                pltpu.async_copy(x_hbm.at[idx_s.at[j]], rows_v, sem).wait()
                pltpu.sync_copy(rows_v, acc_sh.at[idx_d.at[j]], add=True)
                return c2

            lax.fori_loop(0, SUB, step, 0)
            return carry

        lax.fori_loop(0, NSUPER, super_step, 0)
        plsc.subcore_barrier()

        for k in range(RPT // ZCH):
            off = base + k * ZCH
            pltpu.sync_copy(acc_sh.at[pl.ds(off, ZCH), :], stage_v)
            pltpu.sync_copy(stage_v, out_hbm.at[cid, pl.ds(off, ZCH), :])

    return agg


_R = 1000
_RA = 1024


def _norm_body(do0_ref, do1_ref, di0_ref, di1_ref, ns_ref, nd_ref):
    do = do0_ref[...] + do1_ref[...]
    di = di0_ref[...] + di1_ref[...]
    ns_ref[...] = lax.rsqrt(jnp.maximum(do, 1.0))
    nd_ref[...] = lax.rsqrt(jnp.maximum(di, 1.0))


_norm = pl.pallas_call(
    _norm_body,
    out_shape=(jax.ShapeDtypeStruct((1, NP), jnp.float32),
               jax.ShapeDtypeStruct((1, NP), jnp.float32)),
)


def _mm1_body(ns_ref, h_ref, w_ref, o_ref):
    o_ref[...] = jnp.dot(ns_ref[...] * h_ref[...], w_ref[...],
                         preferred_element_type=jnp.float32)


_mm1 = pl.pallas_call(
    _mm1_body,
    grid=(N // _R,),
    in_specs=[
        pl.BlockSpec((_R, 1), lambda i: (i, 0)),
        pl.BlockSpec((_R, F), lambda i: (i, 0)),
        pl.BlockSpec((F, F), lambda i: (0, 0)),
    ],
    out_specs=pl.BlockSpec((_R, F), lambda i: (i, 0)),
    out_shape=jax.ShapeDtypeStruct((N, F), jnp.float32),
)


def _make_layer(dout):
    def body(a0_ref, a1_ref, nd_ref, ns_ref, w_ref, o_ref):
        hp = jnp.maximum(nd_ref[...] * (a0_ref[...] + a1_ref[...]), 0.0)
        o_ref[...] = jnp.dot(ns_ref[...] * hp, w_ref[...],
                             preferred_element_type=jnp.float32)

    return pl.pallas_call(
        body,
        grid=(NA // _RA,),
        in_specs=[
            pl.BlockSpec((_RA, F), lambda i: (i, 0)),
            pl.BlockSpec((_RA, F), lambda i: (i, 0)),
            pl.BlockSpec((_RA, 1), lambda i: (i, 0)),
            pl.BlockSpec((_RA, 1), lambda i: (i, 0)),
            pl.BlockSpec((F, dout), lambda i: (0, 0)),
        ],
        out_specs=pl.BlockSpec((_RA, dout), lambda i: (i, 0)),
        out_shape=jax.ShapeDtypeStruct((NA, dout), jnp.float32),
    )


D3 = 64

_layer128 = _make_layer(F)


def _final_body(a0_ref, a1_ref, nd_ref, o_ref):
    hp = jnp.maximum(nd_ref[...] * (a0_ref[...] + a1_ref[...]), 0.0)
    o_ref[...] = hp[:, :D3]


_final = pl.pallas_call(
    _final_body,
    grid=(NA // _RA,),
    in_specs=[
        pl.BlockSpec((_RA, F), lambda i: (i, 0)),
        pl.BlockSpec((_RA, F), lambda i: (i, 0)),
        pl.BlockSpec((_RA, 1), lambda i: (i, 0)),
    ],
    out_specs=pl.BlockSpec((_RA, D3), lambda i: (i, 0)),
    out_shape=jax.ShapeDtypeStruct((NA, D3), jnp.float32),
)


def kernel(h, edge_index, W1, W2, W3):
    src = edge_index[0].reshape(NW, NSUPER, SUB, CH)
    dst = edge_index[1].reshape(NW, NSUPER, SUB, CH)

    ones_c = jnp.zeros((CH, DW), jnp.float32).at[:, 0].set(1.0)
    zero_c = jnp.zeros((DPT, DW), jnp.float32)
    degp = _make_deg()(src, dst, ones_c, zero_c)
    d0 = degp[..., 0]
    ns_row, nd_row = _norm(d0[0, 0].reshape(1, NP), d0[1, 0].reshape(1, NP),
                           d0[0, 1].reshape(1, NP), d0[1, 1].reshape(1, NP))
    ns = ns_row.reshape(NP, 1)
    nd = nd_row.reshape(NP, 1)

    x1 = _mm1(ns[:N], h, W1)
    a1 = _make_agg(F)(x1, src, dst)
    x2 = _layer128(a1[0], a1[1], nd, ns, W2)
    a2 = _make_agg(F)(x2, src, dst)
    W3p = jnp.pad(W3, ((0, 0), (0, F - C_OUT)))
    x3 = _layer128(a2[0], a2[1], nd, ns, W3p)
    a3 = _make_agg(F)(x3, src, dst)
    out = _final(a3[0], a3[1], nd)
    return out[:N, :C_OUT]


# speedup vs baseline: 8.9178x; 1.2999x over previous
import functools

import jax
import jax.numpy as jnp
from jax import lax
from jax.experimental import pallas as pl
from jax.experimental.pallas import tpu as pltpu
from jax.experimental.pallas import tpu_sc as plsc

N = 10000
E = 320000
F = 128
C_OUT = 40

NC = 2
NS = 16
NW = NC * NS
EPW = E // NW
CH = 100
NCHUNK = EPW // CH
SUB = 20
NSUPER = NCHUNK // SUB
NP = 10240
NA = NP
RPT = NA // NS
ZCH = 64
DPT = NP // NS

@functools.cache
def _mesh():
    return plsc.VectorSubcoreMesh(core_axis_name="c", subcore_axis_name="s",
                                  num_cores=NC, num_subcores=NS)


def _fill_zero_rows(buf, nrows, ncol16):
    zf = jnp.zeros((16,), jnp.float32)

    def body(r, carry):
        for j in range(ncol16):
            buf[r, pl.ds(j * 16, 16)] = zf
        return carry

    lax.fori_loop(0, nrows, body, 0)


DW = 8


@functools.cache
def _make_deg():
    return functools.partial(
        pl.kernel,
        out_type=jax.ShapeDtypeStruct((NC, 2, NP, DW), jnp.float32),
        mesh=_mesh(),
        compiler_params=pltpu.CompilerParams(use_tc_tiling_on_sc=False),
        scratch_types=[
            pltpu.VMEM((SUB, CH), jnp.int32),
            pltpu.VMEM((SUB, CH), jnp.int32),
            pltpu.VMEM((CH, DW), jnp.float32),
            pltpu.VMEM((DPT, DW), jnp.float32),
            pltpu.VMEM_SHARED((NP, DW), jnp.float32),
            pltpu.VMEM_SHARED((NP, DW), jnp.float32),
        ],
    )(_deg_body)


def _deg_body(src_hbm, dst_hbm, ones_hbm, zero_hbm, out_hbm,
              idx_s, idx_d, ones_v, stage_v, dego_sh, degi_sh):
    cid = lax.axis_index("c")
    sid = lax.axis_index("s")
    wid = cid * NS + sid

    pltpu.sync_copy(ones_hbm, ones_v)
    pltpu.sync_copy(zero_hbm, stage_v)

    base = sid * DPT
    pltpu.sync_copy(stage_v, dego_sh.at[pl.ds(base, DPT), :])
    pltpu.sync_copy(stage_v, degi_sh.at[pl.ds(base, DPT), :])
    plsc.subcore_barrier()

    def super_step(g, carry):
        pltpu.sync_copy(src_hbm.at[wid, g], idx_s)
        pltpu.sync_copy(dst_hbm.at[wid, g], idx_d)

        def step(j, c2):
            pltpu.sync_copy(ones_v, dego_sh.at[idx_s.at[j]], add=True)
            pltpu.sync_copy(ones_v, degi_sh.at[idx_d.at[j]], add=True)
            return c2

        lax.fori_loop(0, SUB, step, 0)
        return carry

    lax.fori_loop(0, NSUPER, super_step, 0)
    plsc.subcore_barrier()

    pltpu.sync_copy(dego_sh.at[pl.ds(base, DPT), :], stage_v)
    pltpu.sync_copy(stage_v, out_hbm.at[cid, 0, pl.ds(base, DPT), :])
    pltpu.sync_copy(degi_sh.at[pl.ds(base, DPT), :], stage_v)
    pltpu.sync_copy(stage_v, out_hbm.at[cid, 1, pl.ds(base, DPT), :])


@functools.cache
def _make_agg(d):
    @functools.partial(
        pl.kernel,
        out_type=jax.ShapeDtypeStruct((NC, NA, d), jnp.float32),
        mesh=_mesh(),
        scratch_types=[
            pltpu.VMEM((SUB, CH), jnp.int32),
            pltpu.VMEM((SUB, CH), jnp.int32),
            pltpu.VMEM((CH, d), jnp.float32),
            pltpu.VMEM((CH, d), jnp.float32),
            pltpu.VMEM((ZCH, d), jnp.float32),
            pltpu.VMEM_SHARED((NA, d), jnp.float32),
            pltpu.SemaphoreType.DMA,
            pltpu.SemaphoreType.DMA,
            pltpu.SemaphoreType.DMA,
            pltpu.SemaphoreType.DMA,
        ],
    )
    def agg(x_hbm, src_hbm, dst_hbm, out_hbm, idx_s, idx_d, rows_a, rows_b,
            stage_v, acc_sh, sga, sgb, ssa, ssb):
        cid = lax.axis_index("c")
        sid = lax.axis_index("s")
        wid = cid * NS + sid

        _fill_zero_rows(stage_v, ZCH, d // 16)
        base = sid * RPT
        for k in range(RPT // ZCH):
            pltpu.sync_copy(stage_v, acc_sh.at[pl.ds(base + k * ZCH, ZCH), :])
        plsc.subcore_barrier()

        pairs = SUB // 2

        def super_step(g, carry):
            pltpu.sync_copy(src_hbm.at[wid, g], idx_s)
            pltpu.sync_copy(dst_hbm.at[wid, g], idx_d)
            pltpu.async_copy(x_hbm.at[idx_s.at[0]], rows_a, sga)

            def pair(t, c2):
                ja = 2 * t
                jb = 2 * t + 1
                pltpu.make_async_copy(x_hbm.at[idx_s.at[ja]], rows_a,
                                      sga).wait()

                @pl.when(t > 0)
                def _():
                    pltpu.make_async_copy(rows_b, acc_sh.at[idx_d.at[jb - 2]],
                                          ssb).wait()

                pltpu.async_copy(x_hbm.at[idx_s.at[jb]], rows_b, sgb)
                pltpu.async_copy(rows_a, acc_sh.at[idx_d.at[ja]], ssa,
                                 add=True)
                pltpu.make_async_copy(x_hbm.at[idx_s.at[jb]], rows_b,
                                      sgb).wait()
                pltpu.make_async_copy(rows_a, acc_sh.at[idx_d.at[ja]],
                                      ssa).wait()

                @pl.when(t < pairs - 1)
                def _():
                    pltpu.async_copy(x_hbm.at[idx_s.at[ja + 2]], rows_a, sga)

                pltpu.async_copy(rows_b, acc_sh.at[idx_d.at[jb]], ssb,
                                 add=True)
                return c2

            lax.fori_loop(0, pairs, pair, 0)
            pltpu.make_async_copy(rows_b, acc_sh.at[idx_d.at[SUB - 1]],
                                  ssb).wait()
            return carry

        lax.fori_loop(0, NSUPER, super_step, 0)
        plsc.subcore_barrier()

        for k in range(RPT // ZCH):
            off = base + k * ZCH
            pltpu.sync_copy(acc_sh.at[pl.ds(off, ZCH), :], stage_v)
            pltpu.sync_copy(stage_v, out_hbm.at[cid, pl.ds(off, ZCH), :])

    return agg


_R = 1000
_RA = 1024


def _norm_body(do0_ref, do1_ref, di0_ref, di1_ref, ns_ref, nd_ref):
    do = do0_ref[...] + do1_ref[...]
    di = di0_ref[...] + di1_ref[...]
    ns_ref[...] = lax.rsqrt(jnp.maximum(do, 1.0))
    nd_ref[...] = lax.rsqrt(jnp.maximum(di, 1.0))


_norm = pl.pallas_call(
    _norm_body,
    out_shape=(jax.ShapeDtypeStruct((1, NP), jnp.float32),
               jax.ShapeDtypeStruct((1, NP), jnp.float32)),
)


def _mm1_body(ns_ref, h_ref, w_ref, o_ref):
    o_ref[...] = jnp.dot(ns_ref[...] * h_ref[...], w_ref[...],
                         preferred_element_type=jnp.float32)


_mm1 = pl.pallas_call(
    _mm1_body,
    grid=(N // _R,),
    in_specs=[
        pl.BlockSpec((_R, 1), lambda i: (i, 0)),
        pl.BlockSpec((_R, F), lambda i: (i, 0)),
        pl.BlockSpec((F, F), lambda i: (0, 0)),
    ],
    out_specs=pl.BlockSpec((_R, F), lambda i: (i, 0)),
    out_shape=jax.ShapeDtypeStruct((N, F), jnp.float32),
)


def _make_layer(dout):
    def body(a0_ref, a1_ref, nd_ref, ns_ref, w_ref, o_ref):
        hp = jnp.maximum(nd_ref[...] * (a0_ref[...] + a1_ref[...]), 0.0)
        o_ref[...] = jnp.dot(ns_ref[...] * hp, w_ref[...],
                             preferred_element_type=jnp.float32)

    return pl.pallas_call(
        body,
        grid=(NA // _RA,),
        in_specs=[
            pl.BlockSpec((_RA, F), lambda i: (i, 0)),
            pl.BlockSpec((_RA, F), lambda i: (i, 0)),
            pl.BlockSpec((_RA, 1), lambda i: (i, 0)),
            pl.BlockSpec((_RA, 1), lambda i: (i, 0)),
            pl.BlockSpec((F, dout), lambda i: (0, 0)),
        ],
        out_specs=pl.BlockSpec((_RA, dout), lambda i: (i, 0)),
        out_shape=jax.ShapeDtypeStruct((NA, dout), jnp.float32),
    )


D3 = 64

_layer128 = _make_layer(F)


def _final_body(a0_ref, a1_ref, nd_ref, o_ref):
    hp = jnp.maximum(nd_ref[...] * (a0_ref[...] + a1_ref[...]), 0.0)
    o_ref[...] = hp[:, :D3]


_final = pl.pallas_call(
    _final_body,
    grid=(NA // _RA,),
    in_specs=[
        pl.BlockSpec((_RA, F), lambda i: (i, 0)),
        pl.BlockSpec((_RA, F), lambda i: (i, 0)),
        pl.BlockSpec((_RA, 1), lambda i: (i, 0)),
    ],
    out_specs=pl.BlockSpec((_RA, D3), lambda i: (i, 0)),
    out_shape=jax.ShapeDtypeStruct((NA, D3), jnp.float32),
)


def kernel(h, edge_index, W1, W2, W3):
    src = edge_index[0].reshape(NW, NSUPER, SUB, CH)
    dst = edge_index[1].reshape(NW, NSUPER, SUB, CH)

    ones_c = jnp.zeros((CH, DW), jnp.float32).at[:, 0].set(1.0)
    zero_c = jnp.zeros((DPT, DW), jnp.float32)
    degp = _make_deg()(src, dst, ones_c, zero_c)
    d0 = degp[..., 0]
    ns_row, nd_row = _norm(d0[0, 0].reshape(1, NP), d0[1, 0].reshape(1, NP),
                           d0[0, 1].reshape(1, NP), d0[1, 1].reshape(1, NP))
    ns = ns_row.reshape(NP, 1)
    nd = nd_row.reshape(NP, 1)

    x1 = _mm1(ns[:N], h, W1)
    a1 = _make_agg(F)(x1, src, dst)
    x2 = _layer128(a1[0], a1[1], nd, ns, W2)
    a2 = _make_agg(F)(x2, src, dst)
    W3p = jnp.pad(W3, ((0, 0), (0, F - C_OUT)))
    x3 = _layer128(a2[0], a2[1], nd, ns, W3p)
    a3 = _make_agg(F)(x3, src, dst)
    out = _final(a3[0], a3[1], nd)
    return out[:N, :C_OUT]


# speedup vs baseline: 9.3599x; 1.0496x over previous
import functools

import jax
import jax.numpy as jnp
from jax import lax
from jax.experimental import pallas as pl
from jax.experimental.pallas import tpu as pltpu
from jax.experimental.pallas import tpu_sc as plsc

N = 10000
E = 320000
F = 128
C_OUT = 40

NC = 2
NS = 16
NW = NC * NS
EPW = E // NW
CH = 100
NCHUNK = EPW // CH
SUB = 20
NSUPER = NCHUNK // SUB
NP = 10240
NA = NP
RPT = NA // NS
ZCH = 64
DPT = NP // NS

@functools.cache
def _mesh():
    return plsc.VectorSubcoreMesh(core_axis_name="c", subcore_axis_name="s",
                                  num_cores=NC, num_subcores=NS)


def _fill_zero_rows(buf, nrows, ncol16):
    zf = jnp.zeros((16,), jnp.float32)

    def body(r, carry):
        for j in range(ncol16):
            buf[r, pl.ds(j * 16, 16)] = zf
        return carry

    lax.fori_loop(0, nrows, body, 0)


DW = 8


@functools.cache
def _make_deg():
    return functools.partial(
        pl.kernel,
        out_type=jax.ShapeDtypeStruct((NC, 2, NP, DW), jnp.float32),
        mesh=_mesh(),
        compiler_params=pltpu.CompilerParams(use_tc_tiling_on_sc=False),
        scratch_types=[
            pltpu.VMEM((SUB, CH), jnp.int32),
            pltpu.VMEM((SUB, CH), jnp.int32),
            pltpu.VMEM((CH, DW), jnp.float32),
            pltpu.VMEM((DPT, DW), jnp.float32),
            pltpu.VMEM_SHARED((NP, DW), jnp.float32),
            pltpu.VMEM_SHARED((NP, DW), jnp.float32),
        ],
    )(_deg_body)


def _deg_body(src_hbm, dst_hbm, ones_hbm, zero_hbm, out_hbm,
              idx_s, idx_d, ones_v, stage_v, dego_sh, degi_sh):
    cid = lax.axis_index("c")
    sid = lax.axis_index("s")
    wid = cid * NS + sid

    pltpu.sync_copy(ones_hbm, ones_v)
    pltpu.sync_copy(zero_hbm, stage_v)

    base = sid * DPT
    pltpu.sync_copy(stage_v, dego_sh.at[pl.ds(base, DPT), :])
    pltpu.sync_copy(stage_v, degi_sh.at[pl.ds(base, DPT), :])
    plsc.subcore_barrier()

    def super_step(g, carry):
        pltpu.sync_copy(src_hbm.at[wid, g], idx_s)
        pltpu.sync_copy(dst_hbm.at[wid, g], idx_d)

        def step(j, c2):
            pltpu.sync_copy(ones_v, dego_sh.at[idx_s.at[j]], add=True)
            pltpu.sync_copy(ones_v, degi_sh.at[idx_d.at[j]], add=True)
            return c2

        lax.fori_loop(0, SUB, step, 0)
        return carry

    lax.fori_loop(0, NSUPER, super_step, 0)
    plsc.subcore_barrier()

    pltpu.sync_copy(dego_sh.at[pl.ds(base, DPT), :], stage_v)
    pltpu.sync_copy(stage_v, out_hbm.at[cid, 0, pl.ds(base, DPT), :])
    pltpu.sync_copy(degi_sh.at[pl.ds(base, DPT), :], stage_v)
    pltpu.sync_copy(stage_v, out_hbm.at[cid, 1, pl.ds(base, DPT), :])


@functools.cache
def _make_agg(d, tiled=True):
    @functools.partial(
        pl.kernel,
        out_type=jax.ShapeDtypeStruct((NC, NA, d), jnp.float32),
        mesh=_mesh(),
        compiler_params=pltpu.CompilerParams(use_tc_tiling_on_sc=tiled),
        scratch_types=[
            pltpu.VMEM((SUB, CH), jnp.int32),
            pltpu.VMEM((SUB, CH), jnp.int32),
            pltpu.VMEM((CH, d), jnp.float32),
            pltpu.VMEM((CH, d), jnp.float32),
            pltpu.VMEM((ZCH, d), jnp.float32),
            pltpu.VMEM_SHARED((NA, d), jnp.float32),
            pltpu.SemaphoreType.DMA,
            pltpu.SemaphoreType.DMA,
            pltpu.SemaphoreType.DMA,
            pltpu.SemaphoreType.DMA,
        ],
    )
    def agg(x_hbm, src_hbm, dst_hbm, out_hbm, idx_s, idx_d, rows_a, rows_b,
            stage_v, acc_sh, sga, sgb, ssa, ssb):
        cid = lax.axis_index("c")
        sid = lax.axis_index("s")
        wid = cid * NS + sid

        _fill_zero_rows(stage_v, ZCH, d // 16)
        base = sid * RPT
        for k in range(RPT // ZCH):
            pltpu.sync_copy(stage_v, acc_sh.at[pl.ds(base + k * ZCH, ZCH), :])
        plsc.subcore_barrier()

        pairs = SUB // 2

        def super_step(g, carry):
            pltpu.sync_copy(src_hbm.at[wid, g], idx_s)
            pltpu.sync_copy(dst_hbm.at[wid, g], idx_d)
            pltpu.async_copy(x_hbm.at[idx_s.at[0]], rows_a, sga)

            def pair(t, c2):
                ja = 2 * t
                jb = 2 * t + 1
                pltpu.make_async_copy(x_hbm.at[idx_s.at[ja]], rows_a,
                                      sga).wait()

                @pl.when(t > 0)
                def _():
                    pltpu.make_async_copy(rows_b, acc_sh.at[idx_d.at[jb - 2]],
                                          ssb).wait()

                pltpu.async_copy(x_hbm.at[idx_s.at[jb]], rows_b, sgb)
                pltpu.async_copy(rows_a, acc_sh.at[idx_d.at[ja]], ssa,
                                 add=True)
                pltpu.make_async_copy(x_hbm.at[idx_s.at[jb]], rows_b,
                                      sgb).wait()
                pltpu.make_async_copy(rows_a, acc_sh.at[idx_d.at[ja]],
                                      ssa).wait()

                @pl.when(t < pairs - 1)
                def _():
                    pltpu.async_copy(x_hbm.at[idx_s.at[ja + 2]], rows_a, sga)

                pltpu.async_copy(rows_b, acc_sh.at[idx_d.at[jb]], ssb,
                                 add=True)
                return c2

            lax.fori_loop(0, pairs, pair, 0)
            pltpu.make_async_copy(rows_b, acc_sh.at[idx_d.at[SUB - 1]],
                                  ssb).wait()
            return carry

        lax.fori_loop(0, NSUPER, super_step, 0)
        plsc.subcore_barrier()

        for k in range(RPT // ZCH):
            off = base + k * ZCH
            pltpu.sync_copy(acc_sh.at[pl.ds(off, ZCH), :], stage_v)
            pltpu.sync_copy(stage_v, out_hbm.at[cid, pl.ds(off, ZCH), :])

    return agg


_R = 1000
_RA = 1024


def _norm_body(do0_ref, do1_ref, di0_ref, di1_ref, ns_ref, nd_ref):
    do = do0_ref[...] + do1_ref[...]
    di = di0_ref[...] + di1_ref[...]
    ns_ref[...] = lax.rsqrt(jnp.maximum(do, 1.0))
    nd_ref[...] = lax.rsqrt(jnp.maximum(di, 1.0))


_norm = pl.pallas_call(
    _norm_body,
    out_shape=(jax.ShapeDtypeStruct((1, NP), jnp.float32),
               jax.ShapeDtypeStruct((1, NP), jnp.float32)),
)


def _mm1_body(ns_ref, h_ref, w_ref, o_ref):
    o_ref[...] = jnp.dot(ns_ref[...] * h_ref[...], w_ref[...],
                         preferred_element_type=jnp.float32)


_mm1 = pl.pallas_call(
    _mm1_body,
    grid=(N // _R,),
    in_specs=[
        pl.BlockSpec((_R, 1), lambda i: (i, 0)),
        pl.BlockSpec((_R, F), lambda i: (i, 0)),
        pl.BlockSpec((F, F), lambda i: (0, 0)),
    ],
    out_specs=pl.BlockSpec((_R, F), lambda i: (i, 0)),
    out_shape=jax.ShapeDtypeStruct((N, F), jnp.float32),
)


def _make_layer(dout):
    def body(a0_ref, a1_ref, nd_ref, ns_ref, w_ref, o_ref):
        hp = jnp.maximum(nd_ref[...] * (a0_ref[...] + a1_ref[...]), 0.0)
        o_ref[...] = jnp.dot(ns_ref[...] * hp, w_ref[...],
                             preferred_element_type=jnp.float32)

    return pl.pallas_call(
        body,
        grid=(NA // _RA,),
        in_specs=[
            pl.BlockSpec((_RA, F), lambda i: (i, 0)),
            pl.BlockSpec((_RA, F), lambda i: (i, 0)),
            pl.BlockSpec((_RA, 1), lambda i: (i, 0)),
            pl.BlockSpec((_RA, 1), lambda i: (i, 0)),
            pl.BlockSpec((F, dout), lambda i: (0, 0)),
        ],
        out_specs=pl.BlockSpec((_RA, dout), lambda i: (i, 0)),
        out_shape=jax.ShapeDtypeStruct((NA, dout), jnp.float32),
    )


D3 = 48

_layer128 = _make_layer(F)
_layer48 = _make_layer(D3)


def _final_body(a0_ref, a1_ref, nd_ref, o_ref):
    o_ref[...] = jnp.maximum(nd_ref[...] * (a0_ref[...] + a1_ref[...]), 0.0)


_final = pl.pallas_call(
    _final_body,
    grid=(NA // _RA,),
    in_specs=[
        pl.BlockSpec((_RA, D3), lambda i: (i, 0)),
        pl.BlockSpec((_RA, D3), lambda i: (i, 0)),
        pl.BlockSpec((_RA, 1), lambda i: (i, 0)),
    ],
    out_specs=pl.BlockSpec((_RA, D3), lambda i: (i, 0)),
    out_shape=jax.ShapeDtypeStruct((NA, D3), jnp.float32),
)


def kernel(h, edge_index, W1, W2, W3):
    src = edge_index[0].reshape(NW, NSUPER, SUB, CH)
    dst = edge_index[1].reshape(NW, NSUPER, SUB, CH)

    ones_c = jnp.zeros((CH, DW), jnp.float32).at[:, 0].set(1.0)
    zero_c = jnp.zeros((DPT, DW), jnp.float32)
    degp = _make_deg()(src, dst, ones_c, zero_c)
    d0 = degp[..., 0]
    ns_row, nd_row = _norm(d0[0, 0].reshape(1, NP), d0[1, 0].reshape(1, NP),
                           d0[0, 1].reshape(1, NP), d0[1, 1].reshape(1, NP))
    ns = ns_row.reshape(NP, 1)
    nd = nd_row.reshape(NP, 1)

    x1 = _mm1(ns[:N], h, W1)
    a1 = _make_agg(F)(x1, src, dst)
    x2 = _layer128(a1[0], a1[1], nd, ns, W2)
    a2 = _make_agg(F)(x2, src, dst)
    W3p = jnp.pad(W3, ((0, 0), (0, D3 - C_OUT)))
    x3 = _layer48(a2[0], a2[1], nd, ns, W3p)
    a3 = _make_agg(D3, tiled=False)(x3, src, dst)
    out = _final(a3[0], a3[1], nd)
    return out[:N, :C_OUT]


# speedup vs baseline: 9.4436x; 1.0089x over previous
import functools

import jax
import jax.numpy as jnp
from jax import lax
from jax.experimental import pallas as pl
from jax.experimental.pallas import tpu as pltpu
from jax.experimental.pallas import tpu_sc as plsc

N = 10000
E = 320000
F = 128
C_OUT = 40

NC = 2
NS = 16
NW = NC * NS
EPW = E // NW
CH = 50
NCHUNK = EPW // CH
SUB = 40
NSUPER = NCHUNK // SUB
NP = 10240
NA = NP
RPT = NA // NS
ZCH = 32
DPT = NP // NS

@functools.cache
def _mesh():
    return plsc.VectorSubcoreMesh(core_axis_name="c", subcore_axis_name="s",
                                  num_cores=NC, num_subcores=NS)


def _fill_zero_rows(buf, nrows, ncol16):
    zf = jnp.zeros((16,), jnp.float32)

    def body(r, carry):
        for j in range(ncol16):
            buf[r, pl.ds(j * 16, 16)] = zf
        return carry

    lax.fori_loop(0, nrows, body, 0)


DW = 8


@functools.cache
def _make_deg():
    return functools.partial(
        pl.kernel,
        out_type=jax.ShapeDtypeStruct((NC, 2, NP, DW), jnp.float32),
        mesh=_mesh(),
        compiler_params=pltpu.CompilerParams(use_tc_tiling_on_sc=False),
        scratch_types=[
            pltpu.VMEM((SUB, CH), jnp.int32),
            pltpu.VMEM((SUB, CH), jnp.int32),
            pltpu.VMEM((CH, DW), jnp.float32),
            pltpu.VMEM((DPT, DW), jnp.float32),
            pltpu.VMEM_SHARED((NP, DW), jnp.float32),
            pltpu.VMEM_SHARED((NP, DW), jnp.float32),
        ],
    )(_deg_body)


def _deg_body(src_hbm, dst_hbm, ones_hbm, zero_hbm, out_hbm,
              idx_s, idx_d, ones_v, stage_v, dego_sh, degi_sh):
    cid = lax.axis_index("c")
    sid = lax.axis_index("s")
    wid = cid * NS + sid

    pltpu.sync_copy(ones_hbm, ones_v)
    pltpu.sync_copy(zero_hbm, stage_v)

    base = sid * DPT
    pltpu.sync_copy(stage_v, dego_sh.at[pl.ds(base, DPT), :])
    pltpu.sync_copy(stage_v, degi_sh.at[pl.ds(base, DPT), :])
    plsc.subcore_barrier()

    def super_step(g, carry):
        pltpu.sync_copy(src_hbm.at[wid, g], idx_s)
        pltpu.sync_copy(dst_hbm.at[wid, g], idx_d)

        def step(j, c2):
            pltpu.sync_copy(ones_v, dego_sh.at[idx_s.at[j]], add=True)
            pltpu.sync_copy(ones_v, degi_sh.at[idx_d.at[j]], add=True)
            return c2

        lax.fori_loop(0, SUB, step, 0)
        return carry

    lax.fori_loop(0, NSUPER, super_step, 0)
    plsc.subcore_barrier()

    pltpu.sync_copy(dego_sh.at[pl.ds(base, DPT), :], stage_v)
    pltpu.sync_copy(stage_v, out_hbm.at[cid, 0, pl.ds(base, DPT), :])
    pltpu.sync_copy(degi_sh.at[pl.ds(base, DPT), :], stage_v)
    pltpu.sync_copy(stage_v, out_hbm.at[cid, 1, pl.ds(base, DPT), :])


@functools.cache
def _make_agg(d, tiled=True):
    @functools.partial(
        pl.kernel,
        out_type=jax.ShapeDtypeStruct((NC, NA, d), jnp.float32),
        mesh=_mesh(),
        compiler_params=pltpu.CompilerParams(use_tc_tiling_on_sc=tiled),
        scratch_types=[
            pltpu.VMEM((SUB, CH), jnp.int32),
            pltpu.VMEM((SUB, CH), jnp.int32),
            pltpu.VMEM((CH, d), jnp.float32),
            pltpu.VMEM((CH, d), jnp.float32),
            pltpu.VMEM((CH, d), jnp.float32),
            pltpu.VMEM((CH, d), jnp.float32),
            pltpu.VMEM((ZCH, d), jnp.float32),
            pltpu.VMEM_SHARED((NA, d), jnp.float32),
            [pltpu.SemaphoreType.DMA] * 4,
            [pltpu.SemaphoreType.DMA] * 4,
            pltpu.SemaphoreType.DMA,
        ],
    )
    def agg(x_hbm, src_hbm, dst_hbm, out_hbm, idx_s, idx_d, r0, r1, r2, r3,
            stage_v, acc_sh, sg, ss, sz):
        cid = lax.axis_index("c")
        sid = lax.axis_index("s")
        wid = cid * NS + sid
        rows = (r0, r1, r2, r3)

        _fill_zero_rows(stage_v, ZCH, d // 16)
        base = sid * RPT
        for k in range(RPT // ZCH):
            pltpu.async_copy(stage_v, acc_sh.at[pl.ds(base + k * ZCH, ZCH), :],
                             sz)
        for k in range(RPT // ZCH):
            pltpu.make_async_copy(stage_v,
                                  acc_sh.at[pl.ds(base + k * ZCH, ZCH), :],
                                  sz).wait()
        plsc.subcore_barrier()

        quads = SUB // 4

        def super_step(g, carry):
            pltpu.sync_copy(src_hbm.at[wid, g], idx_s)
            pltpu.sync_copy(dst_hbm.at[wid, g], idx_d)
            pltpu.async_copy(x_hbm.at[idx_s.at[0]], r0, sg[0])
            pltpu.async_copy(x_hbm.at[idx_s.at[1]], r1, sg[1])

            def quad(q, c2):
                j0 = 4 * q
                for i in range(4):
                    j = j0 + i
                    b = i & 3
                    rb = rows[b]
                    pltpu.make_async_copy(x_hbm.at[idx_s.at[j]], rb,
                                          sg[b]).wait()

                    def _wait_prev_scatter():
                        pltpu.make_async_copy(
                            rows[(b + 2) & 3],
                            acc_sh.at[idx_d.at[jnp.maximum(j - 2, 0)]],
                            ss[(b + 2) & 3]).wait()

                    if i >= 2:
                        _wait_prev_scatter()
                    else:
                        pl.when(q > 0)(_wait_prev_scatter)

                    @pl.when(j + 2 < SUB)
                    def _():
                        pltpu.async_copy(x_hbm.at[idx_s.at[j + 2]],
                                         rows[(b + 2) & 3], sg[(b + 2) & 3])

                    pltpu.async_copy(rb, acc_sh.at[idx_d.at[j]], ss[b],
                                     add=True)
                return c2

            lax.fori_loop(0, quads, quad, 0)
            pltpu.make_async_copy(r2, acc_sh.at[idx_d.at[SUB - 2]],
                                  ss[2]).wait()
            pltpu.make_async_copy(r3, acc_sh.at[idx_d.at[SUB - 1]],
                                  ss[3]).wait()
            return carry

        lax.fori_loop(0, NSUPER, super_step, 0)
        plsc.subcore_barrier()

        for k in range(RPT // ZCH):
            off = base + k * ZCH
            pltpu.sync_copy(acc_sh.at[pl.ds(off, ZCH), :], stage_v)
            pltpu.sync_copy(stage_v, out_hbm.at[cid, pl.ds(off, ZCH), :])

    return agg


_R = 1000
_RA = 1024


def _norm_body(do0_ref, do1_ref, di0_ref, di1_ref, ns_ref, nd_ref):
    do = do0_ref[...] + do1_ref[...]
    di = di0_ref[...] + di1_ref[...]
    ns_ref[...] = lax.rsqrt(jnp.maximum(do, 1.0))
    nd_ref[...] = lax.rsqrt(jnp.maximum(di, 1.0))


_norm = pl.pallas_call(
    _norm_body,
    out_shape=(jax.ShapeDtypeStruct((1, NP), jnp.float32),
               jax.ShapeDtypeStruct((1, NP), jnp.float32)),
)


def _mm1_body(ns_ref, h_ref, w_ref, o_ref):
    o_ref[...] = jnp.dot(ns_ref[...] * h_ref[...], w_ref[...],
                         preferred_element_type=jnp.float32)


_mm1 = pl.pallas_call(
    _mm1_body,
    grid=(N // _R,),
    in_specs=[
        pl.BlockSpec((_R, 1), lambda i: (i, 0)),
        pl.BlockSpec((_R, F), lambda i: (i, 0)),
        pl.BlockSpec((F, F), lambda i: (0, 0)),
    ],
    out_specs=pl.BlockSpec((_R, F), lambda i: (i, 0)),
    out_shape=jax.ShapeDtypeStruct((N, F), jnp.float32),
)


def _make_layer(dout):
    def body(a0_ref, a1_ref, nd_ref, ns_ref, w_ref, o_ref):
        hp = jnp.maximum(nd_ref[...] * (a0_ref[...] + a1_ref[...]), 0.0)
        o_ref[...] = jnp.dot(ns_ref[...] * hp, w_ref[...],
                             preferred_element_type=jnp.float32)

    return pl.pallas_call(
        body,
        grid=(NA // _RA,),
        in_specs=[
            pl.BlockSpec((_RA, F), lambda i: (i, 0)),
            pl.BlockSpec((_RA, F), lambda i: (i, 0)),
            pl.BlockSpec((_RA, 1), lambda i: (i, 0)),
            pl.BlockSpec((_RA, 1), lambda i: (i, 0)),
            pl.BlockSpec((F, dout), lambda i: (0, 0)),
        ],
        out_specs=pl.BlockSpec((_RA, dout), lambda i: (i, 0)),
        out_shape=jax.ShapeDtypeStruct((NA, dout), jnp.float32),
    )


D3 = 48

_layer128 = _make_layer(F)
_layer48 = _make_layer(D3)


def _final_body(a0_ref, a1_ref, nd_ref, o_ref):
    o_ref[...] = jnp.maximum(nd_ref[...] * (a0_ref[...] + a1_ref[...]), 0.0)


_final = pl.pallas_call(
    _final_body,
    grid=(NA // _RA,),
    in_specs=[
        pl.BlockSpec((_RA, D3), lambda i: (i, 0)),
        pl.BlockSpec((_RA, D3), lambda i: (i, 0)),
        pl.BlockSpec((_RA, 1), lambda i: (i, 0)),
    ],
    out_specs=pl.BlockSpec((_RA, D3), lambda i: (i, 0)),
    out_shape=jax.ShapeDtypeStruct((NA, D3), jnp.float32),
)


def kernel(h, edge_index, W1, W2, W3):
    src = edge_index[0].reshape(NW, NSUPER, SUB, CH)
    dst = edge_index[1].reshape(NW, NSUPER, SUB, CH)

    ones_c = jnp.zeros((CH, DW), jnp.float32).at[:, 0].set(1.0)
    zero_c = jnp.zeros((DPT, DW), jnp.float32)
    degp = _make_deg()(src, dst, ones_c, zero_c)
    d0 = degp[..., 0]
    ns_row, nd_row = _norm(d0[0, 0].reshape(1, NP), d0[1, 0].reshape(1, NP),
                           d0[0, 1].reshape(1, NP), d0[1, 1].reshape(1, NP))
    ns = ns_row.reshape(NP, 1)
    nd = nd_row.reshape(NP, 1)

    x1 = _mm1(ns[:N], h, W1)
    a1 = _make_agg(F)(x1, src, dst)
    x2 = _layer128(a1[0], a1[1], nd, ns, W2)
    a2 = _make_agg(F)(x2, src, dst)
    W3p = jnp.pad(W3, ((0, 0), (0, D3 - C_OUT)))
    x3 = _layer48(a2[0], a2[1], nd, ns, W3p)
    a3 = _make_agg(D3, tiled=False)(x3, src, dst)
    out = _final(a3[0], a3[1], nd)
    return out[:N, :C_OUT]


# speedup vs baseline: 10.0000x; 1.0589x over previous
import functools

import jax
import jax.numpy as jnp
from jax import lax
from jax.experimental import pallas as pl
from jax.experimental.pallas import tpu as pltpu
from jax.experimental.pallas import tpu_sc as plsc

N = 10000
E = 320000
F = 128
C_OUT = 40

NC = 2
NS = 16
NW = NC * NS
EPW = E // NW
CH = 50
NCHUNK = EPW // CH
SUB = 40
NSUPER = NCHUNK // SUB
CHD = 100
SUBD = 20
NSUPD = EPW // (CHD * SUBD)
NP = 10240
NA = NP
RPT = NA // NS
ZCH = 32
DPT = NP // NS

@functools.cache
def _mesh():
    return plsc.VectorSubcoreMesh(core_axis_name="c", subcore_axis_name="s",
                                  num_cores=NC, num_subcores=NS)


def _fill_zero_rows(buf, nrows, ncol16):
    zf = jnp.zeros((16,), jnp.float32)

    def body(r, carry):
        for j in range(ncol16):
            buf[r, pl.ds(j * 16, 16)] = zf
        return carry

    lax.fori_loop(0, nrows, body, 0)


DW = 8


@functools.cache
def _make_deg():
    return functools.partial(
        pl.kernel,
        out_type=jax.ShapeDtypeStruct((NC, 2, NP, DW), jnp.float32),
        mesh=_mesh(),
        compiler_params=pltpu.CompilerParams(use_tc_tiling_on_sc=False),
        scratch_types=[
            pltpu.VMEM((SUBD, CHD), jnp.int32),
            pltpu.VMEM((SUBD, CHD), jnp.int32),
            pltpu.VMEM((CHD, DW), jnp.float32),
            pltpu.VMEM((DPT, DW), jnp.float32),
            pltpu.VMEM_SHARED((NP, DW), jnp.float32),
            pltpu.VMEM_SHARED((NP, DW), jnp.float32),
            pltpu.SemaphoreType.DMA,
            pltpu.SemaphoreType.DMA,
        ],
    )(_deg_body)


def _deg_body(src_hbm, dst_hbm, ones_hbm, zero_hbm, out_hbm,
              idx_s, idx_d, ones_v, stage_v, dego_sh, degi_sh, so, si):
    cid = lax.axis_index("c")
    sid = lax.axis_index("s")
    wid = cid * NS + sid

    pltpu.sync_copy(ones_hbm, ones_v)
    pltpu.sync_copy(zero_hbm, stage_v)

    base = sid * DPT
    pltpu.sync_copy(stage_v, dego_sh.at[pl.ds(base, DPT), :])
    pltpu.sync_copy(stage_v, degi_sh.at[pl.ds(base, DPT), :])
    plsc.subcore_barrier()

    def super_step(g, carry):
        pltpu.sync_copy(src_hbm.at[wid, g], idx_s)
        pltpu.sync_copy(dst_hbm.at[wid, g], idx_d)

        def fire(j, c2):
            pltpu.async_copy(ones_v, dego_sh.at[idx_s.at[j]], so, add=True)
            pltpu.async_copy(ones_v, degi_sh.at[idx_d.at[j]], si, add=True)
            return c2

        lax.fori_loop(0, SUBD, fire, 0)

        def drain(j, c2):
            pltpu.make_async_copy(ones_v, dego_sh.at[idx_s.at[j]], so).wait()
            pltpu.make_async_copy(ones_v, degi_sh.at[idx_d.at[j]], si).wait()
            return c2

        lax.fori_loop(0, SUBD, drain, 0)
        return carry

    lax.fori_loop(0, NSUPD, super_step, 0)
    plsc.subcore_barrier()

    pltpu.sync_copy(dego_sh.at[pl.ds(base, DPT), :], stage_v)
    pltpu.sync_copy(stage_v, out_hbm.at[cid, 0, pl.ds(base, DPT), :])
    pltpu.sync_copy(degi_sh.at[pl.ds(base, DPT), :], stage_v)
    pltpu.sync_copy(stage_v, out_hbm.at[cid, 1, pl.ds(base, DPT), :])


@functools.cache
def _make_agg(d, tiled=True):
    @functools.partial(
        pl.kernel,
        out_type=jax.ShapeDtypeStruct((NC, NA, d), jnp.float32),
        mesh=_mesh(),
        compiler_params=pltpu.CompilerParams(use_tc_tiling_on_sc=tiled),
        scratch_types=[
            pltpu.VMEM((SUB, CH), jnp.int32),
            pltpu.VMEM((SUB, CH), jnp.int32),
            pltpu.VMEM((CH, d), jnp.float32),
            pltpu.VMEM((CH, d), jnp.float32),
            pltpu.VMEM((CH, d), jnp.float32),
            pltpu.VMEM((CH, d), jnp.float32),
            pltpu.VMEM((ZCH, d), jnp.float32),
            pltpu.VMEM_SHARED((NA, d), jnp.float32),
            [pltpu.SemaphoreType.DMA] * 4,
            [pltpu.SemaphoreType.DMA] * 4,
            pltpu.SemaphoreType.DMA,
        ],
    )
    def agg(x_hbm, src_hbm, dst_hbm, out_hbm, idx_s, idx_d, r0, r1, r2, r3,
            stage_v, acc_sh, sg, ss, sz):
        cid = lax.axis_index("c")
        sid = lax.axis_index("s")
        wid = cid * NS + sid
        rows = (r0, r1, r2, r3)

        _fill_zero_rows(stage_v, ZCH, d // 16)
        base = sid * RPT
        for k in range(RPT // ZCH):
            pltpu.async_copy(stage_v, acc_sh.at[pl.ds(base + k * ZCH, ZCH), :],
                             sz)
        for k in range(RPT // ZCH):
            pltpu.make_async_copy(stage_v,
                                  acc_sh.at[pl.ds(base + k * ZCH, ZCH), :],
                                  sz).wait()
        plsc.subcore_barrier()

        quads = SUB // 4

        def super_step(g, carry):
            pltpu.sync_copy(src_hbm.at[wid, g], idx_s)
            pltpu.sync_copy(dst_hbm.at[wid, g], idx_d)
            pltpu.async_copy(x_hbm.at[idx_s.at[0]], r0, sg[0])
            pltpu.async_copy(x_hbm.at[idx_s.at[1]], r1, sg[1])

            def quad(q, c2):
                j0 = 4 * q
                for i in range(4):
                    j = j0 + i
                    b = i & 3
                    rb = rows[b]
                    pltpu.make_async_copy(x_hbm.at[idx_s.at[j]], rb,
                                          sg[b]).wait()

                    def _wait_prev_scatter():
                        pltpu.make_async_copy(
                            rows[(b + 2) & 3],
                            acc_sh.at[idx_d.at[jnp.maximum(j - 2, 0)]],
                            ss[(b + 2) & 3]).wait()

                    if i >= 2:
                        _wait_prev_scatter()
                    else:
                        pl.when(q > 0)(_wait_prev_scatter)

                    @pl.when(j + 2 < SUB)
                    def _():
                        pltpu.async_copy(x_hbm.at[idx_s.at[j + 2]],
                                         rows[(b + 2) & 3], sg[(b + 2) & 3])

                    pltpu.async_copy(rb, acc_sh.at[idx_d.at[j]], ss[b],
                                     add=True)
                return c2

            lax.fori_loop(0, quads, quad, 0)
            pltpu.make_async_copy(r2, acc_sh.at[idx_d.at[SUB - 2]],
                                  ss[2]).wait()
            pltpu.make_async_copy(r3, acc_sh.at[idx_d.at[SUB - 1]],
                                  ss[3]).wait()
            return carry

        lax.fori_loop(0, NSUPER, super_step, 0)
        plsc.subcore_barrier()

        for k in range(RPT // ZCH):
            off = base + k * ZCH
            pltpu.sync_copy(acc_sh.at[pl.ds(off, ZCH), :], stage_v)
            pltpu.sync_copy(stage_v, out_hbm.at[cid, pl.ds(off, ZCH), :])

    return agg


_R = 1000
_RA = 1024


def _norm_body(do0_ref, do1_ref, di0_ref, di1_ref, ns_ref, nd_ref):
    do = do0_ref[...] + do1_ref[...]
    di = di0_ref[...] + di1_ref[...]
    ns_ref[...] = lax.rsqrt(jnp.maximum(do, 1.0))
    nd_ref[...] = lax.rsqrt(jnp.maximum(di, 1.0))


_norm = pl.pallas_call(
    _norm_body,
    out_shape=(jax.ShapeDtypeStruct((1, NP), jnp.float32),
               jax.ShapeDtypeStruct((1, NP), jnp.float32)),
)


def _mm1_body(ns_ref, h_ref, w_ref, o_ref):
    o_ref[...] = jnp.dot(ns_ref[...] * h_ref[...], w_ref[...],
                         preferred_element_type=jnp.float32)


_mm1 = pl.pallas_call(
    _mm1_body,
    grid=(N // _R,),
    in_specs=[
        pl.BlockSpec((_R, 1), lambda i: (i, 0)),
        pl.BlockSpec((_R, F), lambda i: (i, 0)),
        pl.BlockSpec((F, F), lambda i: (0, 0)),
    ],
    out_specs=pl.BlockSpec((_R, F), lambda i: (i, 0)),
    out_shape=jax.ShapeDtypeStruct((N, F), jnp.float32),
)


def _make_layer(dout):
    def body(a0_ref, a1_ref, nd_ref, ns_ref, w_ref, o_ref):
        hp = jnp.maximum(nd_ref[...] * (a0_ref[...] + a1_ref[...]), 0.0)
        o_ref[...] = jnp.dot(ns_ref[...] * hp, w_ref[...],
                             preferred_element_type=jnp.float32)

    return pl.pallas_call(
        body,
        grid=(NA // _RA,),
        in_specs=[
            pl.BlockSpec((_RA, F), lambda i: (i, 0)),
            pl.BlockSpec((_RA, F), lambda i: (i, 0)),
            pl.BlockSpec((_RA, 1), lambda i: (i, 0)),
            pl.BlockSpec((_RA, 1), lambda i: (i, 0)),
            pl.BlockSpec((F, dout), lambda i: (0, 0)),
        ],
        out_specs=pl.BlockSpec((_RA, dout), lambda i: (i, 0)),
        out_shape=jax.ShapeDtypeStruct((NA, dout), jnp.float32),
    )


D3 = 48

_layer128 = _make_layer(F)
_layer48 = _make_layer(D3)


def _final_body(a0_ref, a1_ref, nd_ref, o_ref):
    o_ref[...] = jnp.maximum(nd_ref[...] * (a0_ref[...] + a1_ref[...]), 0.0)


_final = pl.pallas_call(
    _final_body,
    grid=(NA // _RA,),
    in_specs=[
        pl.BlockSpec((_RA, D3), lambda i: (i, 0)),
        pl.BlockSpec((_RA, D3), lambda i: (i, 0)),
        pl.BlockSpec((_RA, 1), lambda i: (i, 0)),
    ],
    out_specs=pl.BlockSpec((_RA, D3), lambda i: (i, 0)),
    out_shape=jax.ShapeDtypeStruct((NA, D3), jnp.float32),
)


def kernel(h, edge_index, W1, W2, W3):
    src = edge_index[0].reshape(NW, NSUPER, SUB, CH)
    dst = edge_index[1].reshape(NW, NSUPER, SUB, CH)

    ones_c = jnp.zeros((CHD, DW), jnp.float32).at[:, 0].set(1.0)
    zero_c = jnp.zeros((DPT, DW), jnp.float32)
    src_d = edge_index[0].reshape(NW, NSUPD, SUBD, CHD)
    dst_d = edge_index[1].reshape(NW, NSUPD, SUBD, CHD)
    degp = _make_deg()(src_d, dst_d, ones_c, zero_c)
    d0 = degp[..., 0]
    ns_row, nd_row = _norm(d0[0, 0].reshape(1, NP), d0[1, 0].reshape(1, NP),
                           d0[0, 1].reshape(1, NP), d0[1, 1].reshape(1, NP))
    ns = ns_row.reshape(NP, 1)
    nd = nd_row.reshape(NP, 1)

    x1 = _mm1(ns[:N], h, W1)
    a1 = _make_agg(F)(x1, src, dst)
    x2 = _layer128(a1[0], a1[1], nd, ns, W2)
    a2 = _make_agg(F)(x2, src, dst)
    W3p = jnp.pad(W3, ((0, 0), (0, D3 - C_OUT)))
    x3 = _layer48(a2[0], a2[1], nd, ns, W3p)
    a3 = _make_agg(D3, tiled=False)(x3, src, dst)
    out = _final(a3[0], a3[1], nd)
    return out[:N, :C_OUT]
